# baseline (speedup 1.0000x reference)
import functools
import jax
import jax.numpy as jnp
from jax import lax
from jax.experimental import pallas as pl
from jax.experimental.pallas import tpu as pltpu
from jax.experimental.pallas import tpu_sc as plsc

H = 128
NUM_F = 8
CAT_F = 6
BIN_F = 6
TOKENS = NUM_F + CAT_F + BIN_F
CH = 8
L = 16
NJ = H // L
VOCABS = (151, 101, 21, 5, 4, 15)


def kernel(numeric, categorical, binary, W_num, b_num, bin_emb,
           cat_emb_0, cat_emb_1, cat_emb_2, cat_emb_3, cat_emb_4, cat_emb_5):
    B = numeric.shape[0]
    info = plsc.get_sparse_core_info()
    NC, NS = info.num_cores, info.num_subcores
    NW = NC * NS
    rows_w = B // NW
    nch = rows_w // CH

    num_cp = jnp.pad(numeric, ((0, 0), (0, L - NUM_F)))
    num_cp = num_cp.reshape(NW, rows_w * L)

    boffs = (jnp.arange(BIN_F, dtype=jnp.int32) * 2 * H)[None, :]
    idx16 = jnp.concatenate(
        [categorical.astype(jnp.int32) * H,
         binary.astype(jnp.int32) * H + boffs,
         jnp.zeros((B, L - CAT_F - BIN_F), jnp.int32)], axis=1)
    idx_cp = idx16.reshape(NW, rows_w * L)

    cat_tables = [cat_emb_0.reshape(-1), cat_emb_1.reshape(-1),
                  cat_emb_2.reshape(-1), cat_emb_3.reshape(-1),
                  cat_emb_4.reshape(-1), cat_emb_5.reshape(-1)]
    be_flat = bin_emb.reshape(BIN_F * 2 * H)
    mesh = plsc.VectorSubcoreMesh(core_axis_name="c", subcore_axis_name="s")

    @functools.partial(
        pl.kernel, mesh=mesh,
        out_type=jax.ShapeDtypeStruct((B, TOKENS, H), jnp.float32),
        scratch_types=[
            pltpu.VMEM((2, CH, TOKENS, H), jnp.float32),
            pltpu.VMEM((rows_w * L,), jnp.float32),
            pltpu.VMEM((rows_w * L,), jnp.int32),
            pltpu.VMEM((NUM_F, H), jnp.float32),
            pltpu.VMEM((NUM_F, H), jnp.float32),
            pltpu.VMEM((BIN_F * 2 * H,), jnp.float32),
        ] + [pltpu.VMEM((v * H,), jnp.float32) for v in VOCABS]
        + [pltpu.SemaphoreType.DMA],
    )
    def sck(num_hbm, idx_hbm, wn_hbm, bn_hbm, be_hbm,
            ct0_hbm, ct1_hbm, ct2_hbm, ct3_hbm, ct4_hbm, ct5_hbm,
            out_hbm, staging, num_v, idx_v, w_v, b_v, be_v,
            ct0, ct1, ct2, ct3, ct4, ct5, sem):
        wid = lax.axis_index("s") * NC + lax.axis_index("c")
        base = wid * rows_w
        ctabs = [ct0, ct1, ct2, ct3, ct4, ct5]
        ct_hbms = [ct0_hbm, ct1_hbm, ct2_hbm, ct3_hbm, ct4_hbm, ct5_hbm]
        pltpu.sync_copy(num_hbm.at[wid], num_v)
        pltpu.sync_copy(idx_hbm.at[wid], idx_v)
        pltpu.sync_copy(wn_hbm, w_v)
        pltpu.sync_copy(bn_hbm, b_v)
        pltpu.sync_copy(be_hbm, be_v)
        for i in range(CAT_F):
            pltpu.sync_copy(ct_hbms[i], ctabs[i])

        def do_chunk(c, _):
            cmod = c % 2
            sref = staging.at[cmod]

            @pl.when(c >= 2)
            def _drain():
                pltpu.make_async_copy(
                    staging.at[0], out_hbm.at[pl.ds(base, CH)], sem).wait()

            wvecs = [[w_v[t, pl.ds(L * j, L)] for j in range(NJ)]
                     for t in range(NUM_F)]
            bvecs = [[b_v[t, pl.ds(L * j, L)] for j in range(NJ)]
                     for t in range(NUM_F)]

            @plsc.parallel_loop(0, CH, step=1, unroll=CH)
            def _rows(rl):
                iv = idx_v[pl.ds(c * (CH * L) + rl * L, L)]
                xv = num_v[pl.ds(c * (CH * L) + rl * L, L)]
                for t in range(NUM_F):
                    xsp = jnp.full((L,), xv[t], jnp.float32)
                    for j in range(NJ):
                        sref[rl, t, pl.ds(L * j, L)] = (
                            xsp * wvecs[t][j] + bvecs[t][j])
                for i in range(CAT_F):
                    ibase = iv[i]
                    for j in range(NJ):
                        sref[rl, NUM_F + i, pl.ds(L * j, L)] = (
                            ctabs[i][pl.ds(ibase + L * j, L)])
                for i in range(BIN_F):
                    bbase = iv[CAT_F + i]
                    for j in range(NJ):
                        sref[rl, NUM_F + CAT_F + i, pl.ds(L * j, L)] = (
                            be_v[pl.ds(bbase + L * j, L)])

            pltpu.async_copy(
                staging.at[cmod], out_hbm.at[pl.ds(base + c * CH, CH)], sem)
            return 0

        lax.fori_loop(0, nch, do_chunk, 0)
        for _ in range(2):
            pltpu.make_async_copy(
                staging.at[0], out_hbm.at[pl.ds(base, CH)], sem).wait()

    return sck(num_cp, idx_cp, W_num, b_num, be_flat, *cat_tables)


# speedup vs baseline: 1.2198x; 1.2198x over previous
import functools
import jax
import jax.numpy as jnp
from jax import lax
from jax.experimental import pallas as pl
from jax.experimental.pallas import tpu as pltpu
from jax.experimental.pallas import tpu_sc as plsc

H = 128
NUM_F = 8
CAT_F = 6
BIN_F = 6
TOKENS = NUM_F + CAT_F + BIN_F
CH = 8
L = 16
NJ = H // L
VOCABS = (151, 101, 21, 5, 4, 15)


def kernel(numeric, categorical, binary, W_num, b_num, bin_emb,
           cat_emb_0, cat_emb_1, cat_emb_2, cat_emb_3, cat_emb_4, cat_emb_5):
    B = numeric.shape[0]
    info = plsc.get_sparse_core_info()
    NC, NS = info.num_cores, info.num_subcores
    NW = NC * NS
    rows_w = B // NW
    nch = rows_w // CH

    num_cp = jnp.pad(numeric, ((0, 0), (0, L - NUM_F)))
    num_cp = num_cp.reshape(NW, rows_w * L)

    boffs = (jnp.arange(BIN_F, dtype=jnp.int32) * 2 * H)[None, :]
    idx16 = jnp.concatenate(
        [categorical.astype(jnp.int32) * H,
         binary.astype(jnp.int32) * H + boffs,
         jnp.zeros((B, L - CAT_F - BIN_F), jnp.int32)], axis=1)
    idx_cp = idx16.reshape(NW, rows_w * L)

    cat_tables = [cat_emb_0.reshape(-1), cat_emb_1.reshape(-1),
                  cat_emb_2.reshape(-1), cat_emb_3.reshape(-1),
                  cat_emb_4.reshape(-1), cat_emb_5.reshape(-1)]
    be_flat = bin_emb.reshape(BIN_F * 2 * H)
    mesh = plsc.VectorSubcoreMesh(core_axis_name="c", subcore_axis_name="s")

    @functools.partial(
        pl.kernel, mesh=mesh,
        out_type=jax.ShapeDtypeStruct((B, TOKENS, H), jnp.float32),
        scratch_types=[
            pltpu.VMEM((2, CH, TOKENS, H), jnp.float32),
            pltpu.VMEM((rows_w * L,), jnp.float32),
            pltpu.VMEM((rows_w * L,), jnp.int32),
            pltpu.VMEM((NUM_F, H), jnp.float32),
            pltpu.VMEM((NUM_F, H), jnp.float32),
            pltpu.VMEM((BIN_F * 2 * H,), jnp.float32),
        ] + [pltpu.VMEM((v * H,), jnp.float32) for v in VOCABS]
        + [pltpu.SemaphoreType.DMA],
    )
    def sck(num_hbm, idx_hbm, wn_hbm, bn_hbm, be_hbm,
            ct0_hbm, ct1_hbm, ct2_hbm, ct3_hbm, ct4_hbm, ct5_hbm,
            out_hbm, staging, num_v, idx_v, w_v, b_v, be_v,
            ct0, ct1, ct2, ct3, ct4, ct5, sem):
        wid = lax.axis_index("s") * NC + lax.axis_index("c")
        base = wid * rows_w
        ctabs = [ct0, ct1, ct2, ct3, ct4, ct5]
        ct_hbms = [ct0_hbm, ct1_hbm, ct2_hbm, ct3_hbm, ct4_hbm, ct5_hbm]
        pltpu.sync_copy(num_hbm.at[wid], num_v)
        pltpu.sync_copy(idx_hbm.at[wid], idx_v)
        pltpu.sync_copy(wn_hbm, w_v)
        pltpu.sync_copy(bn_hbm, b_v)
        pltpu.sync_copy(be_hbm, be_v)
        for i in range(CAT_F):
            pltpu.sync_copy(ct_hbms[i], ctabs[i])

        def do_chunk(c, _):
            cmod = c % 2
            sref = staging.at[cmod]

            @pl.when(c >= 2)
            def _drain():
                pltpu.make_async_copy(
                    staging.at[0], out_hbm.at[pl.ds(base, CH)], sem).wait()

            wvecs = [[w_v[t, pl.ds(L * j, L)] for j in range(NJ)]
                     for t in range(NUM_F)]
            bvecs = [[b_v[t, pl.ds(L * j, L)] for j in range(NJ)]
                     for t in range(NUM_F)]

            @plsc.parallel_loop(0, CH, step=1, unroll=4)
            def _rows(rl):
                iv = idx_v[pl.ds(c * (CH * L) + rl * L, L)]
                xv = num_v[pl.ds(c * (CH * L) + rl * L, L)]
                for t in range(NUM_F):
                    xsp = jnp.full((L,), xv[t], jnp.float32)
                    for j in range(NJ):
                        sref[rl, t, pl.ds(L * j, L)] = (
                            xsp * wvecs[t][j] + bvecs[t][j])
                for i in range(CAT_F):
                    ibase = iv[i]
                    for j in range(NJ):
                        sref[rl, NUM_F + i, pl.ds(L * j, L)] = (
                            ctabs[i][pl.ds(ibase + L * j, L)])
                for i in range(BIN_F):
                    bbase = iv[CAT_F + i]
                    for j in range(NJ):
                        sref[rl, NUM_F + CAT_F + i, pl.ds(L * j, L)] = (
                            be_v[pl.ds(bbase + L * j, L)])

            pltpu.async_copy(
                staging.at[cmod], out_hbm.at[pl.ds(base + c * CH, CH)], sem)
            return 0

        lax.fori_loop(0, nch, do_chunk, 0)
        for _ in range(2):
            pltpu.make_async_copy(
                staging.at[0], out_hbm.at[pl.ds(base, CH)], sem).wait()

    return sck(num_cp, idx_cp, W_num, b_num, be_flat, *cat_tables)
